# baseline (speedup 1.0000x reference)
import functools

import jax
import jax.numpy as jnp
from jax import lax
from jax.experimental import pallas as pl
from jax.experimental.pallas import tpu as pltpu
from jax.experimental.pallas import tpu_sc as plsc

_EPS = 1e-6
_NW = 32
_CHUNK = 32
_FEAT = 512
_NCLS = 1000
_NEG_INF = float("-inf")


def _store_scalar(ref, i, val, lane):
    idx = jnp.broadcast_to(i, (16,)).astype(jnp.int32)
    v = jnp.broadcast_to(val, (16,))
    plsc.store_scatter(ref, [idx], v, mask=lane == 0)


def _sc_body(xf_hbm, predsf_hbm, lab_hbm, cent_hbm, d2p_hbm, d2n_hbm,
             preds_v, x_v, pos_v, neg_v, lab_v, adv_v, d2p_v, d2n_v,
             sem_p, sem_x, sem_g1, sem_g2, *, rows_per_worker):
    cid = lax.axis_index("c")
    sid = lax.axis_index("s")
    wid = sid * 2 + cid
    nchunks = rows_per_worker // _CHUNK
    lane = lax.broadcasted_iota(jnp.int32, (16,), 0)

    def chunk_body(g, _):
        base = wid * rows_per_worker + g * _CHUNK
        cp_p = pltpu.async_copy(
            predsf_hbm.at[pl.ds(base * _NCLS, _CHUNK * _NCLS)],
            preds_v.at[pl.ds(0, _CHUNK * _NCLS)], sem_p)
        cp_x = pltpu.async_copy(
            xf_hbm.at[pl.ds(base * _FEAT, _CHUNK * _FEAT)], x_v, sem_x)
        pltpu.sync_copy(lab_hbm.at[pl.ds(base, _CHUNK)],
                        lab_v.at[pl.ds(0, _CHUNK)])
        cp_p.wait()

        def am_row(r, _):
            lab = lab_v[pl.ds(r, 16)][0]
            off = r * _NCLS
            cur_max = jnp.full((16,), _NEG_INF, jnp.float32)
            cur_idx = jnp.zeros((16,), jnp.int32)
            for j in range(63):
                pos = lane + j * 16
                v = preds_v[pl.ds(off + j * 16, 16)]
                v = jnp.where((pos == lab) | (pos >= _NCLS), _NEG_INF, v)
                upd = v > cur_max
                cur_idx = jnp.where(upd, pos, cur_idx)
                cur_max = jnp.maximum(v, cur_max)
            gmax = plsc.cummax(cur_max)[15]
            cand_idx = jnp.where(cur_max == gmax, cur_idx, jnp.int32(2 ** 30))
            adv = -plsc.cummax(-cand_idx)[15]
            _store_scalar(adv_v, r, adv, lane)
            return 0

        lax.fori_loop(0, _CHUNK, am_row, 0)

        gp = pltpu.async_copy(cent_hbm.at[lab_v.at[pl.ds(0, _CHUNK)]],
                              pos_v, sem_g1)
        gn = pltpu.async_copy(cent_hbm.at[adv_v], neg_v, sem_g2)
        cp_x.wait()
        gp.wait()
        gn.wait()

        def d2_row(r, _):
            accp = jnp.zeros((16,), jnp.float32)
            accn = jnp.zeros((16,), jnp.float32)
            xoff = r * _FEAT
            for j in range(_FEAT // 16):
                xa = x_v[pl.ds(xoff + j * 16, 16)]
                tp = xa - pos_v[r, pl.ds(j * 16, 16)] + _EPS
                tn = xa - neg_v[r, pl.ds(j * 16, 16)] + _EPS
                accp = accp + tp * tp
                accn = accn + tn * tn
            _store_scalar(d2p_v, r, plsc.cumsum(accp)[15], lane)
            _store_scalar(d2n_v, r, plsc.cumsum(accn)[15], lane)
            return 0

        lax.fori_loop(0, _CHUNK, d2_row, 0)
        pltpu.sync_copy(d2p_v, d2p_hbm.at[pl.ds(base, _CHUNK)])
        pltpu.sync_copy(d2n_v, d2n_hbm.at[pl.ds(base, _CHUNK)])
        return 0

    lax.fori_loop(0, nchunks, chunk_body, 0)


def _finish_kernel(d2p_ref, d2n_ref, out_ref, *, inv_batch):
    d_ap = jnp.sqrt(jnp.maximum(d2p_ref[...], 0.0))
    d_an = jnp.sqrt(jnp.maximum(d2n_ref[...], 0.0))
    out_ref[0, 0] = jnp.sum(jnp.maximum(d_ap - d_an + 1.0, 0.0)) * inv_batch


def kernel(x, preds, labels, centers):
    batch, feat = x.shape
    rows_per_worker = batch // _NW
    lab32 = labels.astype(jnp.int32)
    xf = x.reshape(batch * feat)
    predsf = preds.reshape(batch * _NCLS)

    sc = functools.partial(
        pl.kernel,
        out_type=(jax.ShapeDtypeStruct((batch,), jnp.float32),
                  jax.ShapeDtypeStruct((batch,), jnp.float32)),
        mesh=plsc.VectorSubcoreMesh(core_axis_name="c", subcore_axis_name="s"),
        compiler_params=pltpu.CompilerParams(needs_layout_passes=False),
        scratch_types=[
            pltpu.VMEM((_CHUNK * _NCLS + 16,), jnp.float32),
            pltpu.VMEM((_CHUNK * _FEAT,), jnp.float32),
            pltpu.VMEM((_CHUNK, _FEAT), jnp.float32),
            pltpu.VMEM((_CHUNK, _FEAT), jnp.float32),
            pltpu.VMEM((_CHUNK + 16,), jnp.int32),
            pltpu.VMEM((_CHUNK,), jnp.int32),
            pltpu.VMEM((_CHUNK,), jnp.float32),
            pltpu.VMEM((_CHUNK,), jnp.float32),
            pltpu.SemaphoreType.DMA,
            pltpu.SemaphoreType.DMA,
            pltpu.SemaphoreType.DMA,
            pltpu.SemaphoreType.DMA,
        ],
    )(functools.partial(_sc_body, rows_per_worker=rows_per_worker))
    d2p, d2n = sc(xf, predsf, lab32, centers)

    out = pl.pallas_call(
        functools.partial(_finish_kernel, inv_batch=1.0 / batch),
        in_specs=[
            pl.BlockSpec((8, batch // 8), lambda: (0, 0)),
            pl.BlockSpec((8, batch // 8), lambda: (0, 0)),
        ],
        out_specs=pl.BlockSpec(memory_space=pltpu.SMEM),
        out_shape=jax.ShapeDtypeStruct((1, 1), jnp.float32),
    )(d2p.reshape(8, batch // 8), d2n.reshape(8, batch // 8))
    return out[0, 0]


# speedup vs baseline: 1.3308x; 1.3308x over previous
import functools

import jax
import jax.numpy as jnp
from jax import lax
from jax.experimental import pallas as pl
from jax.experimental.pallas import tpu as pltpu
from jax.experimental.pallas import tpu_sc as plsc

_EPS = 1e-6
_NW = 32
_CHUNK = 32
_FEAT = 512


def _store_scalar(ref, i, val, lane):
    idx = jnp.broadcast_to(i, (16,)).astype(jnp.int32)
    v = jnp.broadcast_to(val, (16,))
    plsc.store_scatter(ref, [idx], v, mask=lane == 0)


def _argmax_kernel(preds_ref, labels_ref, adv_ref):
    preds = preds_ref[...]
    labels = labels_ref[...]
    b, c = preds.shape
    iota = lax.broadcasted_iota(jnp.int32, (b, c), 1)
    masked = jnp.where(iota == labels, -jnp.inf, preds)
    rowmax = jnp.max(masked, axis=1, keepdims=True)
    adv_ref[...] = jnp.min(jnp.where(masked == rowmax, iota, c), axis=1,
                           keepdims=True)


def _sc_body(x_hbm, lab_hbm, adv_hbm, cent_hbm, d2p_hbm, d2n_hbm,
             x_v, pos_v, neg_v, lab_v, adv_v, d2p_v, d2n_v,
             sem_x0, sem_x1, sem_g0, sem_g1, *, rows_per_worker):
    cid = lax.axis_index("c")
    sid = lax.axis_index("s")
    wid = sid * 2 + cid
    lane = lax.broadcasted_iota(jnp.int32, (16,), 0)
    nchunks = rows_per_worker // _CHUNK
    sems_x = (sem_x0, sem_x1)
    sems_g = (sem_g0, sem_g1)

    def start(g, b):
        base = wid * rows_per_worker + g * _CHUNK
        pltpu.sync_copy(lab_hbm.at[pl.ds(base, _CHUNK)], lab_v.at[b])
        pltpu.sync_copy(adv_hbm.at[pl.ds(base, _CHUNK)], adv_v.at[b])
        cp_x = pltpu.async_copy(
            x_hbm.at[pl.ds(base, _CHUNK), :], x_v.at[b], sems_x[b])
        gp = pltpu.async_copy(cent_hbm.at[lab_v.at[b]], pos_v.at[b], sems_g[b])
        gn = pltpu.async_copy(cent_hbm.at[adv_v.at[b]], neg_v.at[b], sems_g[b])
        return cp_x, gp, gn

    def finish(g, b, cp_x, gp, gn):
        base = wid * rows_per_worker + g * _CHUNK
        cp_x.wait()
        gp.wait()
        gn.wait()

        def d2_row(r, _):
            accp = jnp.zeros((16,), jnp.float32)
            accn = jnp.zeros((16,), jnp.float32)
            for j in range(_FEAT // 16):
                xa = x_v.at[b][r, pl.ds(j * 16, 16)]
                tp = xa - pos_v.at[b][r, pl.ds(j * 16, 16)] + _EPS
                tn = xa - neg_v.at[b][r, pl.ds(j * 16, 16)] + _EPS
                accp = accp + tp * tp
                accn = accn + tn * tn
            _store_scalar(d2p_v, r, plsc.cumsum(accp)[15], lane)
            _store_scalar(d2n_v, r, plsc.cumsum(accn)[15], lane)
            return 0

        lax.fori_loop(0, _CHUNK, d2_row, 0)
        pltpu.sync_copy(d2p_v, d2p_hbm.at[pl.ds(base, _CHUNK)])
        pltpu.sync_copy(d2n_v, d2n_hbm.at[pl.ds(base, _CHUNK)])

    hands = [start(0, 0)]
    for g in range(nchunks):
        if g + 1 < nchunks:
            hands.append(start(g + 1, (g + 1) % 2))
        finish(g, g % 2, *hands[g])


def _finish_kernel(d2p_ref, d2n_ref, out_ref, *, inv_batch):
    d_ap = jnp.sqrt(jnp.maximum(d2p_ref[...], 0.0))
    d_an = jnp.sqrt(jnp.maximum(d2n_ref[...], 0.0))
    out_ref[0, 0] = jnp.sum(jnp.maximum(d_ap - d_an + 1.0, 0.0)) * inv_batch


def kernel(x, preds, labels, centers):
    batch, feat = x.shape
    num_classes = centers.shape[0]
    rows_per_worker = batch // _NW
    lab32 = labels.astype(jnp.int32)

    blk = 512
    adv = pl.pallas_call(
        _argmax_kernel,
        grid=(batch // blk,),
        in_specs=[
            pl.BlockSpec((blk, num_classes), lambda i: (i, 0)),
            pl.BlockSpec((blk, 1), lambda i: (i, 0)),
        ],
        out_specs=pl.BlockSpec((blk, 1), lambda i: (i, 0)),
        out_shape=jax.ShapeDtypeStruct((batch, 1), jnp.int32),
    )(preds, lab32.reshape(batch, 1))

    sc = functools.partial(
        pl.kernel,
        out_type=(jax.ShapeDtypeStruct((batch,), jnp.float32),
                  jax.ShapeDtypeStruct((batch,), jnp.float32)),
        mesh=plsc.VectorSubcoreMesh(core_axis_name="c", subcore_axis_name="s"),
        compiler_params=pltpu.CompilerParams(needs_layout_passes=False),
        scratch_types=[
            pltpu.VMEM((2, _CHUNK, _FEAT), jnp.float32),
            pltpu.VMEM((2, _CHUNK, _FEAT), jnp.float32),
            pltpu.VMEM((2, _CHUNK, _FEAT), jnp.float32),
            pltpu.VMEM((2, _CHUNK), jnp.int32),
            pltpu.VMEM((2, _CHUNK), jnp.int32),
            pltpu.VMEM((_CHUNK,), jnp.float32),
            pltpu.VMEM((_CHUNK,), jnp.float32),
            pltpu.SemaphoreType.DMA,
            pltpu.SemaphoreType.DMA,
            pltpu.SemaphoreType.DMA,
            pltpu.SemaphoreType.DMA,
        ],
    )(functools.partial(_sc_body, rows_per_worker=rows_per_worker))
    d2p, d2n = sc(x, lab32, adv.reshape(batch), centers)

    out = pl.pallas_call(
        functools.partial(_finish_kernel, inv_batch=1.0 / batch),
        out_specs=pl.BlockSpec(memory_space=pltpu.SMEM),
        out_shape=jax.ShapeDtypeStruct((1, 1), jnp.float32),
    )(d2p.reshape(8, batch // 8), d2n.reshape(8, batch // 8))
    return out[0, 0]
